# baseline (speedup 1.0000x reference)
import functools
import math

import jax
import jax.numpy as jnp
from jax import lax
from jax.experimental import pallas as pl
from jax.experimental.pallas import tpu as pltpu
from jax.experimental.pallas import tpu_sc as plsc

N_CARDS = 1000
N_ARCHS = 100000
BATCH = 16384

NC = 2
NS = 16
NW = NC * NS
LANES = 16

OUT_CHUNK = 4096


def _sc_gather(idx, rm, tail):
  mesh = plsc.VectorSubcoreMesh(core_axis_name="c", subcore_axis_name="s")

  @functools.partial(
      pl.kernel,
      out_type=jax.ShapeDtypeStruct((N_CARDS, BATCH), jnp.float32),
      mesh=mesh,
      compiler_params=pltpu.CompilerParams(needs_layout_passes=False),
      scratch_types=[
          pltpu.VMEM((99968 + 128,), jnp.float32),
          pltpu.VMEM((BATCH,), jnp.int32),
          pltpu.VMEM((OUT_CHUNK,), jnp.float32),
          pltpu.SemaphoreType.DMA,
      ],
  )
  def k(idx_hbm, rm_hbm, tail_hbm, out_hbm, row_v, idx_v, out_v, sem):
    wid = lax.axis_index("s") * NC + lax.axis_index("c")
    base = wid * 31 + jnp.minimum(wid, 8)
    count = 31 + (wid < 8).astype(jnp.int32)
    pltpu.sync_copy(idx_hbm, idx_v)

    SUBS = ((0, 25088), (25088, 25088), (50176, 25088), (75264, 24704))

    def do_row(r, _):
      c = base + r
      for off, ln in SUBS:
        pltpu.async_copy(rm_hbm.at[c, pl.ds(off, ln)],
                         row_v.at[pl.ds(off, ln)], sem)
      pltpu.async_copy(tail_hbm.at[c], row_v.at[pl.ds(99968, 128)], sem)
      for off, ln in SUBS:
        pltpu.make_async_copy(rm_hbm.at[c, pl.ds(off, ln)],
                              row_v.at[pl.ds(off, ln)], sem).wait()
      pltpu.make_async_copy(tail_hbm.at[c], row_v.at[pl.ds(99968, 128)],
                            sem).wait()

      def do_chunk(kk, _):
        def do_vreg(i, _):
          iv = idx_v[pl.ds(kk * OUT_CHUNK + i * LANES, LANES)]
          out_v[pl.ds(i * LANES, LANES)] = plsc.load_gather(row_v, [iv])
          return 0

        lax.fori_loop(0, OUT_CHUNK // LANES, do_vreg, 0, unroll=8)
        pltpu.sync_copy(out_v, out_hbm.at[c, pl.ds(kk * OUT_CHUNK, OUT_CHUNK)])
        return 0

      lax.fori_loop(0, BATCH // OUT_CHUNK, do_chunk, 0)
      return 0

    lax.fori_loop(0, count, do_row, 0)

  return k(idx, rm, tail)


CB = 128
BB = 2048


def _tc_mul(ranksT, pack):
  def body(rt_ref, p_ref, o_ref):
    o_ref[...] = rt_ref[...].T * p_ref[...]

  return pl.pallas_call(
      body,
      grid=(math.ceil(N_CARDS / CB), BATCH // BB),
      in_specs=[
          pl.BlockSpec((CB, BB), lambda i, j: (i, j)),
          pl.BlockSpec((BB, CB), lambda i, j: (j, i)),
      ],
      out_specs=pl.BlockSpec((BB, CB), lambda i, j: (j, i)),
      out_shape=jax.ShapeDtypeStruct((BATCH, N_CARDS), jnp.float32),
  )(ranksT, pack)


def kernel(x, ranking_matrix):
  idx = x[:, 0].astype(jnp.int32)
  pack = x[:, 1 + N_CARDS:]
  tail = jnp.pad(ranking_matrix[:, N_ARCHS - 32:], ((0, 0), (0, 96)))
  ranksT = _sc_gather(idx, ranking_matrix, tail)
  return _tc_mul(ranksT, pack)
